# baseline (speedup 1.0000x reference)
import functools

import jax
import jax.numpy as jnp
from jax import lax
from jax.experimental import pallas as pl
from jax.experimental.pallas import tpu as pltpu
from jax.experimental.pallas import tpu_sc as plsc

_B = 256
_NEL = 19
_N = _B * _NEL
_D0 = 20000
_WLEN = 25
_NPOOL = 800
_CHUNK = 3200
_NCHUNK = 6
_RB = 2432
_G8 = _N // _RB


def _pool_mats():
    j = jnp.arange(_CHUNK)
    pc = (j[:, None] // _WLEN == jnp.arange(128)[None, :]).astype(jnp.float32) / _WLEN
    jt = jnp.arange(_D0 - _NCHUNK * _CHUNK)
    pt = (jt[:, None] // _WLEN == jnp.arange(128)[None, :]).astype(jnp.float32) / _WLEN
    return pc, pt


def _onehot_el():
    return (jnp.arange(_N)[:, None] % _NEL == jnp.arange(_NEL)[None, :]
            ).astype(jnp.float32)


_KW = 256


def _k1a_body(xt_ref, pc_ref, pt_ref, w2_ref, b2_ref,
              h1_ref, rs_ref, rq_ref, pooled_ref):
    dn = (((0,), (0,)), ((), ()))
    for c in range(_NCHUNK):
        pooled_ref[:, 128 * c:128 * (c + 1)] = lax.dot_general(
            xt_ref[pl.ds(_CHUNK * c, _CHUNK), :], pc_ref[...], dn,
            preferred_element_type=jnp.float32)
    pooled_ref[:, _NCHUNK * 128:(_NCHUNK + 1) * 128] = lax.dot_general(
        xt_ref[pl.ds(_NCHUNK * _CHUNK, _D0 - _NCHUNK * _CHUNK), :], pt_ref[...],
        dn, preferred_element_type=jnp.float32)
    h = jnp.dot(pooled_ref[...], w2_ref[...], preferred_element_type=jnp.float32)
    h = jnp.maximum(h + b2_ref[...], 0.0)
    h1_ref[...] = h
    rs_ref[...] = jnp.sum(h, axis=1, keepdims=True)
    rq_ref[...] = jnp.sum(h * h, axis=1, keepdims=True)


def _stage1(xt, w2pad, b2):
    pc, pt = _pool_mats()
    nblk = _N // _KW
    return pl.pallas_call(
        _k1a_body,
        grid=(nblk,),
        in_specs=[
            pl.BlockSpec((_D0, _KW), lambda i: (0, i)),
            pl.BlockSpec((_CHUNK, 128), lambda i: (0, 0)),
            pl.BlockSpec((_D0 - _NCHUNK * _CHUNK, 128), lambda i: (0, 0)),
            pl.BlockSpec(((_NCHUNK + 1) * 128, 512), lambda i: (0, 0)),
            pl.BlockSpec((1, 512), lambda i: (0, 0)),
        ],
        out_specs=[
            pl.BlockSpec((_KW, 512), lambda i: (i, 0)),
            pl.BlockSpec((_KW, 1), lambda i: (i, 0)),
            pl.BlockSpec((_KW, 1), lambda i: (i, 0)),
        ],
        out_shape=[
            jax.ShapeDtypeStruct((_N, 512), jnp.float32),
            jax.ShapeDtypeStruct((_N, 1), jnp.float32),
            jax.ShapeDtypeStruct((_N, 1), jnp.float32),
        ],
        scratch_shapes=[pltpu.VMEM((_KW, (_NCHUNK + 1) * 128), jnp.float32)],
    )(xt, pc, pt, w2pad, b2)


def _resolve_cols(rs_ref, rq_ref, oh_ref, g_ref, be_ref, count, scc_ref, shc_ref):
    oh = oh_ref[...]
    dn0 = (((0,), (0,)), ((), ()))
    s19 = lax.dot_general(rs_ref[...], oh, dn0, preferred_element_type=jnp.float32)
    q19 = lax.dot_general(rq_ref[...], oh, dn0, preferred_element_type=jnp.float32)
    m = s19 / count
    v = q19 / count - m * m
    inv = lax.rsqrt(v + 1e-5) * g_ref[...]
    sh = be_ref[...] - m * inv
    dn1 = (((1,), (1,)), ((), ()))
    scc_ref[...] = lax.dot_general(oh, inv, dn1, preferred_element_type=jnp.float32)
    shc_ref[...] = lax.dot_general(oh, sh, dn1, preferred_element_type=jnp.float32)


def _mk_mlp_bn(din, dout, count, wexp=False):
    def body(*refs):
        if wexp:
            (h_ref, rs_ref, rq_ref, oh_ref, g_ref, be_ref, w_ref, b_ref,
             ew1_ref, ew2_ref, out_ref, rso_ref, rqo_ref, w1t_ref, w2t_ref,
             scc_ref, shc_ref) = refs
        else:
            (h_ref, rs_ref, rq_ref, oh_ref, g_ref, be_ref, w_ref, b_ref,
             out_ref, rso_ref, rqo_ref, scc_ref, shc_ref) = refs
        i = pl.program_id(0)

        @pl.when(i == 0)
        def _resolve():
            _resolve_cols(rs_ref, rq_ref, oh_ref, g_ref, be_ref, count,
                          scc_ref, shc_ref)
            if wexp:
                for ew_ref, wt_ref in ((ew1_ref, w1t_ref), (ew2_ref, w2t_ref)):
                    ew = ew_ref[...]
                    sp = jnp.maximum(ew, 0.0) + jnp.log1p(jnp.exp(-jnp.abs(ew)))
                    wt_ref[...] = jnp.broadcast_to(sp, (_B, 60))

        off = pl.multiple_of(i * _RB, _RB)
        sc = scc_ref[pl.ds(off, _RB), :]
        sh = shc_ref[pl.ds(off, _RB), :]
        hb = h_ref[...] * sc + sh
        h2 = jnp.dot(hb, w_ref[...], preferred_element_type=jnp.float32)
        h2 = jnp.maximum(h2 + b_ref[...], 0.0)
        out_ref[...] = h2
        rso_ref[...] = jnp.sum(h2, axis=1, keepdims=True)
        rqo_ref[...] = jnp.sum(h2 * h2, axis=1, keepdims=True)

    full = lambda shape: pl.BlockSpec(shape, lambda i: tuple(0 for _ in shape))
    out_shape = [
        jax.ShapeDtypeStruct((_N, dout), jnp.float32),
        jax.ShapeDtypeStruct((_N, 1), jnp.float32),
        jax.ShapeDtypeStruct((_N, 1), jnp.float32),
    ]
    out_specs = [
        pl.BlockSpec((_RB, dout), lambda i: (i, 0)),
        pl.BlockSpec((_RB, 1), lambda i: (i, 0)),
        pl.BlockSpec((_RB, 1), lambda i: (i, 0)),
    ]
    if wexp:
        out_shape += [jax.ShapeDtypeStruct((_B, 60), jnp.float32)] * 2
        out_specs += [full((_B, 60))] * 2

    def call(h, rs, rq, g, be, w, b, *ews):
        return pl.pallas_call(
            body,
            grid=(_G8,),
            in_specs=[
                pl.BlockSpec((_RB, din), lambda i: (i, 0)),
                full((_N, 1)), full((_N, 1)), full((_N, _NEL)),
                full((1, _NEL)), full((1, _NEL)),
                full((din, dout)), full((1, dout)),
            ] + ([full((1, 60))] * 2 if wexp else []),
            out_specs=out_specs,
            out_shape=out_shape,
            scratch_shapes=[pltpu.VMEM((_N, 1), jnp.float32),
                            pltpu.VMEM((_N, 1), jnp.float32)],
        )(h, rs, rq, _onehot_el(), g.reshape(1, _NEL), be.reshape(1, _NEL),
          w, b, *ews)

    return call


def _mk_root(din, dmid):
    def body(h_ref, wr_ref, root_ref):
        root_ref[...] = jnp.dot(h_ref[:, :din], wr_ref[...],
                                preferred_element_type=jnp.float32)

    def call(h, wroot):
        return pl.pallas_call(
            body,
            grid=(_G8,),
            in_specs=[
                pl.BlockSpec((_RB, h.shape[1]), lambda i: (i, 0)),
                pl.BlockSpec((din, dmid), lambda i: (0, 0)),
            ],
            out_specs=pl.BlockSpec((_RB, dmid), lambda i: (i, 0)),
            out_shape=jax.ShapeDtypeStruct((_N, dmid), jnp.float32),
        )(h, wroot)

    return call


def _mk_apply_root(din, dmid, col_scale, dpad=None):
    dpad = dpad or din

    def body(h_ref, a_ref, b_ref, oh_ref, g_ref, be_ref, wr_ref,
             hn_ref, root_ref, scc_ref, shc_ref):
        i = pl.program_id(0)
        if col_scale:
            @pl.when(i == 0)
            def _resolve():
                _resolve_cols(a_ref, b_ref, oh_ref, g_ref, be_ref,
                              _B * float(din), scc_ref, shc_ref)

            off = pl.multiple_of(i * _RB, _RB)
            sc = scc_ref[pl.ds(off, _RB), :]
            sh = shc_ref[pl.ds(off, _RB), :]
        else:
            m = a_ref[...] / float(_N)
            v = b_ref[...] / float(_N) - m * m
            sc = lax.rsqrt(v + 1e-5) * g_ref[...]
            sh = be_ref[...] - m * sc
        hn = h_ref[...] * sc + sh
        if dpad > din:
            hn_ref[:, :din] = hn
            hn_ref[:, din:] = jnp.zeros((_RB, dpad - din), jnp.float32)
        else:
            hn_ref[...] = hn
        root_ref[...] = jnp.dot(hn, wr_ref[...], preferred_element_type=jnp.float32)

    full = lambda shape: pl.BlockSpec(shape, lambda i: tuple(0 for _ in shape))
    sspec = (full((_N, 1)) if col_scale else full((1, din)))
    gspec = (full((1, _NEL)) if col_scale else full((1, din)))

    def call(h, a, b, g, be, wroot):
        return pl.pallas_call(
            body,
            grid=(_G8,),
            in_specs=[
                pl.BlockSpec((_RB, din), lambda i: (i, 0)),
                sspec, sspec, full((_N, _NEL)), gspec, gspec,
                full((din, dmid)),
            ],
            out_specs=[
                pl.BlockSpec((_RB, dpad), lambda i: (i, 0)),
                pl.BlockSpec((_RB, dmid), lambda i: (i, 0)),
            ],
            out_shape=[
                jax.ShapeDtypeStruct((_N, dpad), jnp.float32),
                jax.ShapeDtypeStruct((_N, dmid), jnp.float32),
            ],
            scratch_shapes=[pltpu.VMEM((_N, 1), jnp.float32),
                            pltpu.VMEM((_N, 1), jnp.float32)],
        )(h, a, b, _onehot_el(),
          g.reshape(1, -1) if g.ndim == 1 else g,
          be.reshape(1, -1) if be.ndim == 1 else be, wroot)

    return call


def _mk_combine(din, dout, pool_out=False):
    def body(p_ref, root_ref, wrel_ref, brel_ref, out_ref, *rest):
        if pool_out:
            mn_ref, s_ref, q_ref = rest
        else:
            s_ref, q_ref = rest
        agg = p_ref[0] + p_ref[1]
        h = jnp.dot(agg, wrel_ref[...], preferred_element_type=jnp.float32)
        h = jnp.maximum(h + brel_ref[...] + root_ref[...], 0.0)
        if pool_out:
            h3 = h.reshape(_RB // _NEL, _NEL, dout)
            out_ref[...] = jnp.max(h3, axis=1)
            mn_ref[...] = jnp.min(h3, axis=1)
        else:
            out_ref[...] = h
        ps = jnp.sum(h, axis=0).reshape(1, dout)
        pq = jnp.sum(h * h, axis=0).reshape(1, dout)

        @pl.when(pl.program_id(0) == 0)
        def _init():
            s_ref[...] = jnp.zeros_like(s_ref)
            q_ref[...] = jnp.zeros_like(q_ref)

        s_ref[...] += ps
        q_ref[...] += pq

    hspec = (pl.BlockSpec((_RB // _NEL, dout), lambda i: (i, 0)) if pool_out
             else pl.BlockSpec((_RB, dout), lambda i: (i, 0)))
    hshape = ((_B, dout) if pool_out else (_N, dout))
    out_specs = [hspec] + ([hspec] if pool_out else []) + [
        pl.BlockSpec((1, dout), lambda i: (0, 0)),
        pl.BlockSpec((1, dout), lambda i: (0, 0)),
    ]
    out_shape = ([jax.ShapeDtypeStruct(hshape, jnp.float32)] *
                 (2 if pool_out else 1)) + [
        jax.ShapeDtypeStruct((1, dout), jnp.float32),
        jax.ShapeDtypeStruct((1, dout), jnp.float32),
    ]

    def call(parts, root, wrel, brel):
        return pl.pallas_call(
            body,
            grid=(_G8,),
            in_specs=[
                pl.BlockSpec((2, _RB, din), lambda i: (0, i, 0)),
                pl.BlockSpec((_RB, dout), lambda i: (i, 0)),
                pl.BlockSpec((din, dout), lambda i: (0, 0)),
                pl.BlockSpec((1, dout), lambda i: (0, 0)),
            ],
            out_specs=out_specs,
            out_shape=out_shape,
        )(parts, root, wrel, brel)

    return call


_NEDGE_TOT = 15360
_NWORK = 32
_EPW = _NEDGE_TOT // _NWORK
_ECH = 96
_NCH = _EPW // _ECH
_RPW = _N // 16


def _mk_gconv_sc(d):
    mesh = plsc.VectorSubcoreMesh(core_axis_name="c", subcore_axis_name="s")

    @functools.partial(
        pl.kernel,
        out_type=jax.ShapeDtypeStruct((2, _N, d), jnp.float32),
        mesh=mesh,
        scratch_types=[
            pltpu.VMEM((_EPW,), jnp.int32),
            pltpu.VMEM((_NCH, _ECH), jnp.int32),
            pltpu.VMEM((_EPW,), jnp.float32),
            pltpu.VMEM((_EPW, d), jnp.float32),
            pltpu.VMEM_SHARED((_N, d), jnp.float32),
            pltpu.SemaphoreType.DMA,
            pltpu.SemaphoreType.DMA,
        ],
    )
    def k(h_hbm, src_hbm, dst_hbm, w_hbm, z_hbm, out_hbm,
          sidx, didx, wv, rows, agg, sem, zsem):
        c = lax.axis_index("c")
        s = lax.axis_index("s")
        wid = s * 2 + c
        base = wid * _EPW
        zcp = pltpu.async_copy(z_hbm.at[pl.ds(s * _RPW, _RPW)],
                               agg.at[pl.ds(s * _RPW, _RPW)], zsem)
        pltpu.sync_copy(src_hbm.at[pl.ds(base, _EPW)], sidx)
        for j in range(_NCH):
            pltpu.sync_copy(dst_hbm.at[pl.ds(base + _ECH * j, _ECH)], didx.at[j])
        pltpu.sync_copy(w_hbm.at[pl.ds(base, _EPW)], wv)
        cps = [pltpu.async_copy(h_hbm.at[sidx.at[pl.ds(_ECH * j, _ECH)]],
                                rows.at[pl.ds(_ECH * j, _ECH)], sem)
               for j in range(_NCH)]

        gpc = _ECH // 16

        def scale(g, carry):
            wg = wv[pl.ds(g * 16, 16)]
            for i in range(16):
                e = g * 16 + i
                wvec = jnp.full((16,), 0.0, jnp.float32) + wg[i]
                for q in range(d // 16):
                    sl = pl.ds(q * 16, 16)
                    rows[e, sl] = rows[e, sl] * wvec
            return carry

        for j in range(_NCH):
            cps[j].wait()
            lax.fori_loop(j * gpc, (j + 1) * gpc, scale, 0)
        zcp.wait()
        plsc.subcore_barrier()
        scps = [pltpu.async_copy(rows.at[pl.ds(_ECH * j, _ECH)],
                                 agg.at[didx.at[j]], sem, add=True)
                for j in range(_NCH)]
        for cp in scps:
            cp.wait()
        plsc.subcore_barrier()
        pltpu.sync_copy(agg.at[pl.ds(s * _RPW, _RPW)],
                        out_hbm.at[c].at[pl.ds(s * _RPW, _RPW)])

    return k


def _heads_body(mx_ref, mn_ref, s_ref, q_ref, g_ref, be_ref,
                w5_ref, b5_ref, wc_ref, bc_ref, out_ref):
    m = s_ref[...] / float(_N)
    v = q_ref[...] / float(_N) - m * m
    sc = lax.rsqrt(v + 1e-5) * g_ref[...]
    sh = be_ref[...] - m * sc
    pooled = jnp.where(sc >= 0.0, mx_ref[...] * sc, mn_ref[...] * sc) + sh
    feat = jnp.dot(pooled, w5_ref[...], preferred_element_type=jnp.float32)
    feat = jnp.maximum(feat + b5_ref[...], 0.0)
    z = jnp.dot(feat, wc_ref[...], preferred_element_type=jnp.float32) + bc_ref[...]
    ps = []
    for k in range(3):
        zp = z[:, 2 * k:2 * k + 2]
        m = jnp.max(zp, axis=1, keepdims=True)
        e = jnp.exp(zp - m)
        ps.append(e / jnp.sum(e, axis=1, keepdims=True))
    p0, p1, p2 = ps
    p_hc = p0[:, 0:1] * p1[:, 0:1]
    p_ad = p0[:, 1:2] * p2[:, 1:2]
    p_ftd = p0[:, 0:1] * p1[:, 1:2] + p0[:, 1:2] * p2[:, 0:1]
    out_ref[...] = jnp.log(jnp.concatenate([p_hc, p_ftd, p_ad], axis=1) + 1e-8)


def _heads(mx, mn, s5, q5, g7, be7, w5, b5, wcat, bcat):
    return pl.pallas_call(
        _heads_body,
        out_shape=jax.ShapeDtypeStruct((_B, 3), jnp.float32),
    )(mx, mn, s5, q5, g7.reshape(1, 64), be7.reshape(1, 64), w5, b5, wcat, bcat)


def kernel(x, edge_index, batch, W2, b2, g3, be3, W3, b3, g4, be4, W4, b4,
           g5, be5, ew1, Wrel1, brel1, Wroot1, g6, be6, ew2, Wrel2, brel2,
           Wroot2, g7, be7, W5, b5, Whr, bhr, Whf, bhf, Wfa, bfa):
    w2pad = jnp.concatenate(
        [W2, jnp.zeros(((_NCHUNK + 1) * 128 - _NPOOL, 512), jnp.float32)], axis=0)
    h1, rs1, rq1 = _stage1(jnp.swapaxes(x, 0, 1), w2pad, b2.reshape(1, 512))
    h2, rs2, rq2, w1t, w2t = _mk_mlp_bn(512, 256, _B * 512.0, wexp=True)(
        h1, rs1, rq1, g3, be3, W3, b3.reshape(1, 256),
        ew1.reshape(1, 60), ew2.reshape(1, 60))
    h3, rs3, rq3 = _mk_mlp_bn(256, 128, _B * 256.0)(
        h2, rs2, rq2, g4, be4, W4, b4.reshape(1, 128))
    h3n, _unused1 = _mk_apply_root(128, 2, True)(h3, rs3, rq3, g5, be5,
                                                 jnp.zeros((128, 2), jnp.float32))

    src = edge_index[0]
    dst = edge_index[1]
    w1e = w1t.reshape(_NEDGE_TOT)
    w2e = w2t.reshape(_NEDGE_TOT)

    z128 = jnp.zeros((_N, 128), jnp.float32)
    parts1 = _mk_gconv_sc(128)(h3n, src, dst, w1e, z128)
    root1 = _mk_root(128, 64)(h3n, Wroot1)
    h4, s4, q4 = _mk_combine(128, 64)(parts1, root1, Wrel1, brel1.reshape(1, 64))
    h4n, _unused2 = _mk_apply_root(64, 2, False, dpad=128)(
        h4, s4, q4, g6.reshape(1, 64), be6.reshape(1, 64),
        jnp.zeros((64, 2), jnp.float32))

    parts2 = _mk_gconv_sc(128)(h4n, src, dst, w2e, z128)
    wroot2p = jnp.concatenate([Wroot2, jnp.zeros((64, 64), jnp.float32)], axis=0)
    root2 = _mk_root(128, 64)(h4n, wroot2p)
    wrel2p = jnp.concatenate([Wrel2, jnp.zeros((64, 64), jnp.float32)], axis=0)
    mx, mn, s5, q5 = _mk_combine(128, 64, pool_out=True)(
        parts2, root2, wrel2p, brel2.reshape(1, 64))
    wcat = jnp.concatenate([Whr, Whf, Wfa], axis=1)
    bcat = jnp.concatenate([bhr, bhf, bfa]).reshape(1, 6)
    return _heads(mx, mn, s5, q5, g7, be7, W5, b5.reshape(1, 32), wcat, bcat)


# speedup vs baseline: 1.0095x; 1.0095x over previous
import functools

import jax
import jax.numpy as jnp
from jax import lax
from jax.experimental import pallas as pl
from jax.experimental.pallas import tpu as pltpu
from jax.experimental.pallas import tpu_sc as plsc

_B = 256
_NEL = 19
_N = _B * _NEL
_D0 = 20000
_WLEN = 25
_NPOOL = 800
_CHUNK = 3200
_NCHUNK = 6
_RB = 2432
_G8 = _N // _RB


def _pool_mats():
    j = jnp.arange(_CHUNK)
    pc = (j[:, None] // _WLEN == jnp.arange(128)[None, :]).astype(jnp.float32) / _WLEN
    jt = jnp.arange(_D0 - _NCHUNK * _CHUNK)
    pt = (jt[:, None] // _WLEN == jnp.arange(128)[None, :]).astype(jnp.float32) / _WLEN
    return pc, pt


def _onehot_el():
    return (jnp.arange(_N)[:, None] % _NEL == jnp.arange(_NEL)[None, :]
            ).astype(jnp.float32)


_KW = 256


def _k1a_body(xt_ref, pc_ref, pt_ref, w2_ref, b2_ref,
              h1_ref, rs_ref, rq_ref, pooled_ref):
    dn = (((0,), (0,)), ((), ()))
    for c in range(_NCHUNK):
        pooled_ref[:, 128 * c:128 * (c + 1)] = lax.dot_general(
            xt_ref[pl.ds(_CHUNK * c, _CHUNK), :], pc_ref[...], dn,
            preferred_element_type=jnp.float32)
    pooled_ref[:, _NCHUNK * 128:(_NCHUNK + 1) * 128] = lax.dot_general(
        xt_ref[pl.ds(_NCHUNK * _CHUNK, _D0 - _NCHUNK * _CHUNK), :], pt_ref[...],
        dn, preferred_element_type=jnp.float32)
    h = jnp.dot(pooled_ref[...], w2_ref[...], preferred_element_type=jnp.float32)
    h = jnp.maximum(h + b2_ref[...], 0.0)
    h1_ref[...] = h
    rs_ref[...] = jnp.sum(h, axis=1, keepdims=True)
    rq_ref[...] = jnp.sum(h * h, axis=1, keepdims=True)


def _stage1(xt, w2pad, b2):
    pc, pt = _pool_mats()
    nblk = _N // _KW
    return pl.pallas_call(
        _k1a_body,
        grid=(nblk,),
        in_specs=[
            pl.BlockSpec((_D0, _KW), lambda i: (0, i)),
            pl.BlockSpec((_CHUNK, 128), lambda i: (0, 0)),
            pl.BlockSpec((_D0 - _NCHUNK * _CHUNK, 128), lambda i: (0, 0)),
            pl.BlockSpec(((_NCHUNK + 1) * 128, 512), lambda i: (0, 0)),
            pl.BlockSpec((1, 512), lambda i: (0, 0)),
        ],
        out_specs=[
            pl.BlockSpec((_KW, 512), lambda i: (i, 0)),
            pl.BlockSpec((_KW, 1), lambda i: (i, 0)),
            pl.BlockSpec((_KW, 1), lambda i: (i, 0)),
        ],
        out_shape=[
            jax.ShapeDtypeStruct((_N, 512), jnp.float32),
            jax.ShapeDtypeStruct((_N, 1), jnp.float32),
            jax.ShapeDtypeStruct((_N, 1), jnp.float32),
        ],
        scratch_shapes=[pltpu.VMEM((_KW, (_NCHUNK + 1) * 128), jnp.float32)],
    )(xt, pc, pt, w2pad, b2)


def _resolve_cols(rs_ref, rq_ref, oh_ref, g_ref, be_ref, count, scc_ref, shc_ref):
    oh = oh_ref[...]
    dn0 = (((0,), (0,)), ((), ()))
    s19 = lax.dot_general(rs_ref[...], oh, dn0, preferred_element_type=jnp.float32)
    q19 = lax.dot_general(rq_ref[...], oh, dn0, preferred_element_type=jnp.float32)
    m = s19 / count
    v = q19 / count - m * m
    inv = lax.rsqrt(v + 1e-5) * g_ref[...]
    sh = be_ref[...] - m * inv
    dn1 = (((1,), (1,)), ((), ()))
    scc_ref[...] = lax.dot_general(oh, inv, dn1, preferred_element_type=jnp.float32)
    shc_ref[...] = lax.dot_general(oh, sh, dn1, preferred_element_type=jnp.float32)


def _mk_mlp_bn(din, dout, count, wexp=False):
    def body(*refs):
        if wexp:
            (h_ref, rs_ref, rq_ref, oh_ref, g_ref, be_ref, w_ref, b_ref,
             ew1_ref, ew2_ref, out_ref, rso_ref, rqo_ref, w1t_ref, w2t_ref,
             scc_ref, shc_ref) = refs
        else:
            (h_ref, rs_ref, rq_ref, oh_ref, g_ref, be_ref, w_ref, b_ref,
             out_ref, rso_ref, rqo_ref, scc_ref, shc_ref) = refs
        i = pl.program_id(0)

        @pl.when(i == 0)
        def _resolve():
            _resolve_cols(rs_ref, rq_ref, oh_ref, g_ref, be_ref, count,
                          scc_ref, shc_ref)
            if wexp:
                for ew_ref, wt_ref in ((ew1_ref, w1t_ref), (ew2_ref, w2t_ref)):
                    ew = ew_ref[...]
                    sp = jnp.maximum(ew, 0.0) + jnp.log1p(jnp.exp(-jnp.abs(ew)))
                    wt_ref[...] = jnp.broadcast_to(sp, (_B, 60))

        off = pl.multiple_of(i * _RB, _RB)
        sc = scc_ref[pl.ds(off, _RB), :]
        sh = shc_ref[pl.ds(off, _RB), :]
        hb = h_ref[...] * sc + sh
        h2 = jnp.dot(hb, w_ref[...], preferred_element_type=jnp.float32)
        h2 = jnp.maximum(h2 + b_ref[...], 0.0)
        out_ref[...] = h2
        rso_ref[...] = jnp.sum(h2, axis=1, keepdims=True)
        rqo_ref[...] = jnp.sum(h2 * h2, axis=1, keepdims=True)

    full = lambda shape: pl.BlockSpec(shape, lambda i: tuple(0 for _ in shape))
    out_shape = [
        jax.ShapeDtypeStruct((_N, dout), jnp.float32),
        jax.ShapeDtypeStruct((_N, 1), jnp.float32),
        jax.ShapeDtypeStruct((_N, 1), jnp.float32),
    ]
    out_specs = [
        pl.BlockSpec((_RB, dout), lambda i: (i, 0)),
        pl.BlockSpec((_RB, 1), lambda i: (i, 0)),
        pl.BlockSpec((_RB, 1), lambda i: (i, 0)),
    ]
    if wexp:
        out_shape += [jax.ShapeDtypeStruct((_B, 60), jnp.float32)] * 2
        out_specs += [full((_B, 60))] * 2

    def call(h, rs, rq, g, be, w, b, *ews):
        return pl.pallas_call(
            body,
            grid=(_G8,),
            in_specs=[
                pl.BlockSpec((_RB, din), lambda i: (i, 0)),
                full((_N, 1)), full((_N, 1)), full((_N, _NEL)),
                full((1, _NEL)), full((1, _NEL)),
                full((din, dout)), full((1, dout)),
            ] + ([full((1, 60))] * 2 if wexp else []),
            out_specs=out_specs,
            out_shape=out_shape,
            scratch_shapes=[pltpu.VMEM((_N, 1), jnp.float32),
                            pltpu.VMEM((_N, 1), jnp.float32)],
        )(h, rs, rq, _onehot_el(), g.reshape(1, _NEL), be.reshape(1, _NEL),
          w, b, *ews)

    return call


def _mk_apply_root(din, dmid, col_scale, dpad=None):
    dpad = dpad or din

    def body(h_ref, a_ref, b_ref, oh_ref, g_ref, be_ref, wr_ref,
             hn_ref, root_ref, scc_ref, shc_ref):
        i = pl.program_id(0)
        if col_scale:
            @pl.when(i == 0)
            def _resolve():
                _resolve_cols(a_ref, b_ref, oh_ref, g_ref, be_ref,
                              _B * float(din), scc_ref, shc_ref)

            off = pl.multiple_of(i * _RB, _RB)
            sc = scc_ref[pl.ds(off, _RB), :]
            sh = shc_ref[pl.ds(off, _RB), :]
        else:
            m = a_ref[...] / float(_N)
            v = b_ref[...] / float(_N) - m * m
            sc = lax.rsqrt(v + 1e-5) * g_ref[...]
            sh = be_ref[...] - m * sc
        hn = h_ref[...] * sc + sh
        if dpad > din:
            hn_ref[:, :din] = hn
            hn_ref[:, din:] = jnp.zeros((_RB, dpad - din), jnp.float32)
        else:
            hn_ref[...] = hn
        root_ref[...] = jnp.dot(hn, wr_ref[...], preferred_element_type=jnp.float32)

    full = lambda shape: pl.BlockSpec(shape, lambda i: tuple(0 for _ in shape))
    sspec = (full((_N, 1)) if col_scale else full((1, din)))
    gspec = (full((1, _NEL)) if col_scale else full((1, din)))

    def call(h, a, b, g, be, wroot):
        return pl.pallas_call(
            body,
            grid=(_G8,),
            in_specs=[
                pl.BlockSpec((_RB, din), lambda i: (i, 0)),
                sspec, sspec, full((_N, _NEL)), gspec, gspec,
                full((din, dmid)),
            ],
            out_specs=[
                pl.BlockSpec((_RB, dpad), lambda i: (i, 0)),
                pl.BlockSpec((_RB, dmid), lambda i: (i, 0)),
            ],
            out_shape=[
                jax.ShapeDtypeStruct((_N, dpad), jnp.float32),
                jax.ShapeDtypeStruct((_N, dmid), jnp.float32),
            ],
            scratch_shapes=[pltpu.VMEM((_N, 1), jnp.float32),
                            pltpu.VMEM((_N, 1), jnp.float32)],
        )(h, a, b, _onehot_el(),
          g.reshape(1, -1) if g.ndim == 1 else g,
          be.reshape(1, -1) if be.ndim == 1 else be, wroot)

    return call


def _mk_combine(din, dout, pool_out=False):
    def body(p_ref, root_ref, wrel_ref, brel_ref, out_ref, *rest):
        if pool_out:
            mn_ref, s_ref, q_ref = rest
        else:
            s_ref, q_ref = rest
        agg = p_ref[0] + p_ref[1]
        h = jnp.dot(agg, wrel_ref[...], preferred_element_type=jnp.float32)
        h = jnp.maximum(h + brel_ref[...] + root_ref[...], 0.0)
        if pool_out:
            h3 = h.reshape(_RB // _NEL, _NEL, dout)
            out_ref[...] = jnp.max(h3, axis=1)
            mn_ref[...] = jnp.min(h3, axis=1)
        else:
            out_ref[...] = h
        ps = jnp.sum(h, axis=0).reshape(1, dout)
        pq = jnp.sum(h * h, axis=0).reshape(1, dout)

        @pl.when(pl.program_id(0) == 0)
        def _init():
            s_ref[...] = jnp.zeros_like(s_ref)
            q_ref[...] = jnp.zeros_like(q_ref)

        s_ref[...] += ps
        q_ref[...] += pq

    hspec = (pl.BlockSpec((_RB // _NEL, dout), lambda i: (i, 0)) if pool_out
             else pl.BlockSpec((_RB, dout), lambda i: (i, 0)))
    hshape = ((_B, dout) if pool_out else (_N, dout))
    out_specs = [hspec] + ([hspec] if pool_out else []) + [
        pl.BlockSpec((1, dout), lambda i: (0, 0)),
        pl.BlockSpec((1, dout), lambda i: (0, 0)),
    ]
    out_shape = ([jax.ShapeDtypeStruct(hshape, jnp.float32)] *
                 (2 if pool_out else 1)) + [
        jax.ShapeDtypeStruct((1, dout), jnp.float32),
        jax.ShapeDtypeStruct((1, dout), jnp.float32),
    ]

    def call(parts, root, wrel, brel):
        return pl.pallas_call(
            body,
            grid=(_G8,),
            in_specs=[
                pl.BlockSpec((2, _RB, din), lambda i: (0, i, 0)),
                pl.BlockSpec((_RB, dout), lambda i: (i, 0)),
                pl.BlockSpec((din, dout), lambda i: (0, 0)),
                pl.BlockSpec((1, dout), lambda i: (0, 0)),
            ],
            out_specs=out_specs,
            out_shape=out_shape,
        )(parts, root, wrel, brel)

    return call


_NEDGE_TOT = 15360
_NWORK = 32
_EPW = _NEDGE_TOT // _NWORK
_ECH = 96
_NCH = _EPW // _ECH
_RPW = _N // 16


def _mk_gconv_sc(d):
    mesh = plsc.VectorSubcoreMesh(core_axis_name="c", subcore_axis_name="s")

    @functools.partial(
        pl.kernel,
        out_type=jax.ShapeDtypeStruct((2, _N, d), jnp.float32),
        mesh=mesh,
        scratch_types=[
            pltpu.VMEM((_EPW,), jnp.int32),
            pltpu.VMEM((_NCH, _ECH), jnp.int32),
            pltpu.VMEM((_EPW,), jnp.float32),
            pltpu.VMEM((_EPW, d), jnp.float32),
            pltpu.VMEM_SHARED((_N, d), jnp.float32),
            pltpu.SemaphoreType.DMA,
            pltpu.SemaphoreType.DMA,
        ],
    )
    def k(h_hbm, src_hbm, dst_hbm, w_hbm, z_hbm, out_hbm,
          sidx, didx, wv, rows, agg, sem, zsem):
        c = lax.axis_index("c")
        s = lax.axis_index("s")
        wid = s * 2 + c
        base = wid * _EPW
        zcp = pltpu.async_copy(z_hbm.at[pl.ds(s * _RPW, _RPW)],
                               agg.at[pl.ds(s * _RPW, _RPW)], zsem)
        pltpu.sync_copy(src_hbm.at[pl.ds(base, _EPW)], sidx)
        for j in range(_NCH):
            pltpu.sync_copy(dst_hbm.at[pl.ds(base + _ECH * j, _ECH)], didx.at[j])
        pltpu.sync_copy(w_hbm.at[pl.ds(base, _EPW)], wv)
        cps = [pltpu.async_copy(h_hbm.at[sidx.at[pl.ds(_ECH * j, _ECH)]],
                                rows.at[pl.ds(_ECH * j, _ECH)], sem)
               for j in range(_NCH)]

        gpc = _ECH // 16

        def scale(g, carry):
            wg = wv[pl.ds(g * 16, 16)]
            for i in range(16):
                e = g * 16 + i
                wvec = jnp.full((16,), 0.0, jnp.float32) + wg[i]
                for q in range(d // 16):
                    sl = pl.ds(q * 16, 16)
                    rows[e, sl] = rows[e, sl] * wvec
            return carry

        for j in range(_NCH):
            cps[j].wait()
            lax.fori_loop(j * gpc, (j + 1) * gpc, scale, 0)
        zcp.wait()
        plsc.subcore_barrier()
        scps = [pltpu.async_copy(rows.at[pl.ds(_ECH * j, _ECH)],
                                 agg.at[didx.at[j]], sem, add=True)
                for j in range(_NCH)]
        for cp in scps:
            cp.wait()
        plsc.subcore_barrier()
        pltpu.sync_copy(agg.at[pl.ds(s * _RPW, _RPW)],
                        out_hbm.at[c].at[pl.ds(s * _RPW, _RPW)])

    return k


def _heads_body(mx_ref, mn_ref, s_ref, q_ref, g_ref, be_ref,
                w5_ref, b5_ref, wc_ref, bc_ref, out_ref):
    m = s_ref[...] / float(_N)
    v = q_ref[...] / float(_N) - m * m
    sc = lax.rsqrt(v + 1e-5) * g_ref[...]
    sh = be_ref[...] - m * sc
    pooled = jnp.where(sc >= 0.0, mx_ref[...] * sc, mn_ref[...] * sc) + sh
    feat = jnp.dot(pooled, w5_ref[...], preferred_element_type=jnp.float32)
    feat = jnp.maximum(feat + b5_ref[...], 0.0)
    z = jnp.dot(feat, wc_ref[...], preferred_element_type=jnp.float32) + bc_ref[...]
    ps = []
    for k in range(3):
        zp = z[:, 2 * k:2 * k + 2]
        m = jnp.max(zp, axis=1, keepdims=True)
        e = jnp.exp(zp - m)
        ps.append(e / jnp.sum(e, axis=1, keepdims=True))
    p0, p1, p2 = ps
    p_hc = p0[:, 0:1] * p1[:, 0:1]
    p_ad = p0[:, 1:2] * p2[:, 1:2]
    p_ftd = p0[:, 0:1] * p1[:, 1:2] + p0[:, 1:2] * p2[:, 0:1]
    out_ref[...] = jnp.log(jnp.concatenate([p_hc, p_ftd, p_ad], axis=1) + 1e-8)


def _heads(mx, mn, s5, q5, g7, be7, w5, b5, wcat, bcat):
    return pl.pallas_call(
        _heads_body,
        out_shape=jax.ShapeDtypeStruct((_B, 3), jnp.float32),
    )(mx, mn, s5, q5, g7.reshape(1, 64), be7.reshape(1, 64), w5, b5, wcat, bcat)


def kernel(x, edge_index, batch, W2, b2, g3, be3, W3, b3, g4, be4, W4, b4,
           g5, be5, ew1, Wrel1, brel1, Wroot1, g6, be6, ew2, Wrel2, brel2,
           Wroot2, g7, be7, W5, b5, Whr, bhr, Whf, bhf, Wfa, bfa):
    w2pad = jnp.concatenate(
        [W2, jnp.zeros(((_NCHUNK + 1) * 128 - _NPOOL, 512), jnp.float32)], axis=0)
    h1, rs1, rq1 = _stage1(jnp.swapaxes(x, 0, 1), w2pad, b2.reshape(1, 512))
    h2, rs2, rq2, w1t, w2t = _mk_mlp_bn(512, 256, _B * 512.0, wexp=True)(
        h1, rs1, rq1, g3, be3, W3, b3.reshape(1, 256),
        ew1.reshape(1, 60), ew2.reshape(1, 60))
    h3, rs3, rq3 = _mk_mlp_bn(256, 128, _B * 256.0)(
        h2, rs2, rq2, g4, be4, W4, b4.reshape(1, 128))
    h3n, root1 = _mk_apply_root(128, 64, True)(h3, rs3, rq3, g5, be5, Wroot1)

    src = edge_index[0]
    dst = edge_index[1]
    w1e = w1t.reshape(_NEDGE_TOT)
    w2e = w2t.reshape(_NEDGE_TOT)

    z128 = jnp.zeros((_N, 128), jnp.float32)
    parts1 = _mk_gconv_sc(128)(h3n, src, dst, w1e, z128)
    h4, s4, q4 = _mk_combine(128, 64)(parts1, root1, Wrel1, brel1.reshape(1, 64))
    h4n, root2 = _mk_apply_root(64, 64, False, dpad=128)(
        h4, s4, q4, g6.reshape(1, 64), be6.reshape(1, 64), Wroot2)

    parts2 = _mk_gconv_sc(128)(h4n, src, dst, w2e, z128)
    wrel2p = jnp.concatenate([Wrel2, jnp.zeros((64, 64), jnp.float32)], axis=0)
    mx, mn, s5, q5 = _mk_combine(128, 64, pool_out=True)(
        parts2, root2, wrel2p, brel2.reshape(1, 64))
    wcat = jnp.concatenate([Whr, Whf, Wfa], axis=1)
    bcat = jnp.concatenate([bhr, bhf, bfa]).reshape(1, 6)
    return _heads(mx, mn, s5, q5, g7, be7, W5, b5.reshape(1, 32), wcat, bcat)
